# baseline (speedup 1.0000x reference)
import jax
import jax.numpy as jnp
from jax.experimental import pallas as pl
from jax.experimental.pallas import tpu as pltpu

_ROWS, _F = 32768, 256
_NSPLITS = 17

_CHUNKS = (1024, 15360, 15360, 1024)
_N = len(_CHUNKS)
_OFFS = []
_o = 0
for _c in _CHUNKS:
    _OFFS.append(_o)
    _o += _c
assert _o == _ROWS


def _copy_body(flat_ref, cu_ref, out_ref, out_cu_ref, buf, in_sems, out_sems, cu_sem):
    cu = pltpu.make_async_copy(cu_ref, out_cu_ref, cu_sem)
    cu.start()

    def mk_in(i):
        return pltpu.make_async_copy(
            flat_ref.at[pl.ds(_OFFS[i], _CHUNKS[i])],
            buf.at[pl.ds(_OFFS[i], _CHUNKS[i])],
            in_sems.at[i],
        )

    def mk_out(i):
        return pltpu.make_async_copy(
            buf.at[pl.ds(_OFFS[i], _CHUNKS[i])],
            out_ref.at[pl.ds(_OFFS[i], _CHUNKS[i])],
            out_sems.at[i],
        )

    ins = [mk_in(i) for i in range(_N)]
    for c in ins:
        c.start()
    outs = []
    for i in range(_N):
        ins[i].wait()
        c = mk_out(i)
        c.start()
        outs.append(c)
    for c in outs:
        c.wait()
    cu.wait()


def kernel(flat, cu_seqlens):
    return pl.pallas_call(
        _copy_body,
        out_shape=(
            jax.ShapeDtypeStruct((_ROWS, _F), jnp.float32),
            jax.ShapeDtypeStruct((_NSPLITS,), jnp.int32),
        ),
        in_specs=[
            pl.BlockSpec(memory_space=pl.ANY),
            pl.BlockSpec(memory_space=pl.ANY),
        ],
        out_specs=(
            pl.BlockSpec(memory_space=pl.ANY),
            pl.BlockSpec(memory_space=pl.ANY),
        ),
        scratch_shapes=[
            pltpu.VMEM((_ROWS, _F), jnp.float32),
            pltpu.SemaphoreType.DMA((_N,)),
            pltpu.SemaphoreType.DMA((_N,)),
            pltpu.SemaphoreType.DMA,
        ],
    )(flat, cu_seqlens)


# speedup vs baseline: 1.0355x; 1.0355x over previous
import jax
import jax.numpy as jnp
from jax.experimental import pallas as pl
from jax.experimental.pallas import tpu as pltpu

_ROWS, _F = 32768, 256
_NSPLITS = 17

_CHUNKS = (512, 2048, 13824, 13824, 2048, 512)
_N = len(_CHUNKS)
_OFFS = []
_o = 0
for _c in _CHUNKS:
    _OFFS.append(_o)
    _o += _c
assert _o == _ROWS


def _copy_body(flat_ref, cu_ref, out_ref, out_cu_ref, buf, in_sems, out_sems, cu_sem):
    cu = pltpu.make_async_copy(cu_ref, out_cu_ref, cu_sem)
    cu.start()

    def mk_in(i):
        return pltpu.make_async_copy(
            flat_ref.at[pl.ds(_OFFS[i], _CHUNKS[i])],
            buf.at[pl.ds(_OFFS[i], _CHUNKS[i])],
            in_sems.at[i],
        )

    def mk_out(i):
        return pltpu.make_async_copy(
            buf.at[pl.ds(_OFFS[i], _CHUNKS[i])],
            out_ref.at[pl.ds(_OFFS[i], _CHUNKS[i])],
            out_sems.at[i],
        )

    ins = [mk_in(i) for i in range(_N)]
    for c in ins:
        c.start()
    outs = []
    for i in range(_N):
        ins[i].wait()
        c = mk_out(i)
        c.start()
        outs.append(c)
    for c in outs:
        c.wait()
    cu.wait()


def kernel(flat, cu_seqlens):
    return pl.pallas_call(
        _copy_body,
        out_shape=(
            jax.ShapeDtypeStruct((_ROWS, _F), jnp.float32),
            jax.ShapeDtypeStruct((_NSPLITS,), jnp.int32),
        ),
        in_specs=[
            pl.BlockSpec(memory_space=pl.ANY),
            pl.BlockSpec(memory_space=pl.ANY),
        ],
        out_specs=(
            pl.BlockSpec(memory_space=pl.ANY),
            pl.BlockSpec(memory_space=pl.ANY),
        ),
        scratch_shapes=[
            pltpu.VMEM((_ROWS, _F), jnp.float32),
            pltpu.SemaphoreType.DMA((_N,)),
            pltpu.SemaphoreType.DMA((_N,)),
            pltpu.SemaphoreType.DMA,
        ],
    )(flat, cu_seqlens)


# speedup vs baseline: 1.0388x; 1.0032x over previous
import jax
import jax.numpy as jnp
from jax.experimental import pallas as pl
from jax.experimental.pallas import tpu as pltpu

_ROWS, _F = 32768, 256
_NSPLITS = 17

_CHUNKS = (512, 2048, 16384, 11264, 2048, 512)
_N = len(_CHUNKS)
_OFFS = []
_o = 0
for _c in _CHUNKS:
    _OFFS.append(_o)
    _o += _c
assert _o == _ROWS


def _copy_body(flat_ref, cu_ref, out_ref, out_cu_ref, buf, in_sems, out_sems, cu_sem):
    cu = pltpu.make_async_copy(cu_ref, out_cu_ref, cu_sem)
    cu.start()

    def mk_in(i):
        return pltpu.make_async_copy(
            flat_ref.at[pl.ds(_OFFS[i], _CHUNKS[i])],
            buf.at[pl.ds(_OFFS[i], _CHUNKS[i])],
            in_sems.at[i],
        )

    def mk_out(i):
        return pltpu.make_async_copy(
            buf.at[pl.ds(_OFFS[i], _CHUNKS[i])],
            out_ref.at[pl.ds(_OFFS[i], _CHUNKS[i])],
            out_sems.at[i],
        )

    ins = [mk_in(i) for i in range(_N)]
    for c in ins:
        c.start()
    outs = []
    for i in range(_N):
        ins[i].wait()
        c = mk_out(i)
        c.start()
        outs.append(c)
    for c in outs:
        c.wait()
    cu.wait()


def kernel(flat, cu_seqlens):
    return pl.pallas_call(
        _copy_body,
        out_shape=(
            jax.ShapeDtypeStruct((_ROWS, _F), jnp.float32),
            jax.ShapeDtypeStruct((_NSPLITS,), jnp.int32),
        ),
        in_specs=[
            pl.BlockSpec(memory_space=pl.ANY),
            pl.BlockSpec(memory_space=pl.ANY),
        ],
        out_specs=(
            pl.BlockSpec(memory_space=pl.ANY),
            pl.BlockSpec(memory_space=pl.ANY),
        ),
        scratch_shapes=[
            pltpu.VMEM((_ROWS, _F), jnp.float32),
            pltpu.SemaphoreType.DMA((_N,)),
            pltpu.SemaphoreType.DMA((_N,)),
            pltpu.SemaphoreType.DMA,
        ],
    )(flat, cu_seqlens)


# speedup vs baseline: 1.0422x; 1.0032x over previous
import jax
import jax.numpy as jnp
from jax.experimental import pallas as pl
from jax.experimental.pallas import tpu as pltpu

_ROWS, _F = 32768, 256
_NSPLITS = 17

_CHUNKS = (512, 2048, 18432, 9216, 2048, 512)
_N = len(_CHUNKS)
_OFFS = []
_o = 0
for _c in _CHUNKS:
    _OFFS.append(_o)
    _o += _c
assert _o == _ROWS


def _copy_body(flat_ref, cu_ref, out_ref, out_cu_ref, buf, in_sems, out_sems, cu_sem):
    cu = pltpu.make_async_copy(cu_ref, out_cu_ref, cu_sem)
    cu.start()

    def mk_in(i):
        return pltpu.make_async_copy(
            flat_ref.at[pl.ds(_OFFS[i], _CHUNKS[i])],
            buf.at[pl.ds(_OFFS[i], _CHUNKS[i])],
            in_sems.at[i],
        )

    def mk_out(i):
        return pltpu.make_async_copy(
            buf.at[pl.ds(_OFFS[i], _CHUNKS[i])],
            out_ref.at[pl.ds(_OFFS[i], _CHUNKS[i])],
            out_sems.at[i],
        )

    ins = [mk_in(i) for i in range(_N)]
    for c in ins:
        c.start()
    outs = []
    for i in range(_N):
        ins[i].wait()
        c = mk_out(i)
        c.start()
        outs.append(c)
    for c in outs:
        c.wait()
    cu.wait()


def kernel(flat, cu_seqlens):
    return pl.pallas_call(
        _copy_body,
        out_shape=(
            jax.ShapeDtypeStruct((_ROWS, _F), jnp.float32),
            jax.ShapeDtypeStruct((_NSPLITS,), jnp.int32),
        ),
        in_specs=[
            pl.BlockSpec(memory_space=pl.ANY),
            pl.BlockSpec(memory_space=pl.ANY),
        ],
        out_specs=(
            pl.BlockSpec(memory_space=pl.ANY),
            pl.BlockSpec(memory_space=pl.ANY),
        ),
        scratch_shapes=[
            pltpu.VMEM((_ROWS, _F), jnp.float32),
            pltpu.SemaphoreType.DMA((_N,)),
            pltpu.SemaphoreType.DMA((_N,)),
            pltpu.SemaphoreType.DMA,
        ],
    )(flat, cu_seqlens)


# speedup vs baseline: 1.0499x; 1.0074x over previous
import jax
import jax.numpy as jnp
from jax.experimental import pallas as pl
from jax.experimental.pallas import tpu as pltpu

_ROWS, _F = 32768, 256
_NSPLITS = 17

_CHUNKS = (512, 2048, 20480, 7168, 2048, 512)
_N = len(_CHUNKS)
_OFFS = []
_o = 0
for _c in _CHUNKS:
    _OFFS.append(_o)
    _o += _c
assert _o == _ROWS


def _copy_body(flat_ref, cu_ref, out_ref, out_cu_ref, buf, in_sems, out_sems, cu_sem):
    cu = pltpu.make_async_copy(cu_ref, out_cu_ref, cu_sem)
    cu.start()

    def mk_in(i):
        return pltpu.make_async_copy(
            flat_ref.at[pl.ds(_OFFS[i], _CHUNKS[i])],
            buf.at[pl.ds(_OFFS[i], _CHUNKS[i])],
            in_sems.at[i],
        )

    def mk_out(i):
        return pltpu.make_async_copy(
            buf.at[pl.ds(_OFFS[i], _CHUNKS[i])],
            out_ref.at[pl.ds(_OFFS[i], _CHUNKS[i])],
            out_sems.at[i],
        )

    ins = [mk_in(i) for i in range(_N)]
    for c in ins:
        c.start()
    outs = []
    for i in range(_N):
        ins[i].wait()
        c = mk_out(i)
        c.start()
        outs.append(c)
    for c in outs:
        c.wait()
    cu.wait()


def kernel(flat, cu_seqlens):
    return pl.pallas_call(
        _copy_body,
        out_shape=(
            jax.ShapeDtypeStruct((_ROWS, _F), jnp.float32),
            jax.ShapeDtypeStruct((_NSPLITS,), jnp.int32),
        ),
        in_specs=[
            pl.BlockSpec(memory_space=pl.ANY),
            pl.BlockSpec(memory_space=pl.ANY),
        ],
        out_specs=(
            pl.BlockSpec(memory_space=pl.ANY),
            pl.BlockSpec(memory_space=pl.ANY),
        ),
        scratch_shapes=[
            pltpu.VMEM((_ROWS, _F), jnp.float32),
            pltpu.SemaphoreType.DMA((_N,)),
            pltpu.SemaphoreType.DMA((_N,)),
            pltpu.SemaphoreType.DMA,
        ],
    )(flat, cu_seqlens)
